# baseline (speedup 1.0000x reference)
import dataclasses
import functools

import jax
import jax.numpy as jnp
from jax import lax
from jax.experimental import pallas as pl
from jax.experimental.pallas import tpu as pltpu
from jax.experimental.pallas import tpu_sc as plsc

_NC = 2
_NS = 16
_NW = _NC * _NS
_L = 16


def _edge_dot_kernel(n_edges, d_feat, block_w):
    epw = n_edges // _NW
    nb = epw // block_w

    mesh = plsc.VectorSubcoreMesh(core_axis_name="c", subcore_axis_name="s")
    cp = pltpu.CompilerParams()
    if "needs_layout_passes" in pltpu.CompilerParams.__dataclass_fields__:
        cp = dataclasses.replace(cp, needs_layout_passes=False)

    @functools.partial(
        pl.kernel,
        mesh=mesh,
        compiler_params=cp,
        out_type=jax.ShapeDtypeStruct((n_edges,), jnp.float32),
        scratch_types=[
            pltpu.VMEM((epw,), jnp.int32),
            pltpu.VMEM((epw,), jnp.int32),
            pltpu.VMEM((epw,), jnp.float32),
            pltpu.VMEM((block_w, d_feat), jnp.float32),
            pltpu.VMEM((block_w, d_feat), jnp.float32),
            pltpu.VMEM((block_w, d_feat), jnp.float32),
            pltpu.VMEM((block_w, d_feat), jnp.float32),
            pltpu.VMEM((_L, _L + 1), jnp.float32),
            pltpu.SemaphoreType.DMA,
            pltpu.SemaphoreType.DMA,
        ],
    )
    def kern(h_hbm, src_hbm, dst_hbm, out_hbm,
             si_v, di_v, sc_v, hu0, hv0, hu1, hv1, tmp_v, sem0, sem1):
        wid = lax.axis_index("s") * _NC + lax.axis_index("c")
        base = wid * epw
        pltpu.sync_copy(src_hbm.at[pl.ds(base, epw)], si_v)
        pltpu.sync_copy(dst_hbm.at[pl.ds(base, epw)], di_v)

        def fire(b, hu, hv, sem):
            off = b * block_w
            pltpu.async_copy(h_hbm.at[si_v.at[pl.ds(off, block_w)]], hu, sem)
            pltpu.async_copy(h_hbm.at[di_v.at[pl.ds(off, block_w)]], hv, sem)

        def drain(b, hu, hv, sem):
            off = b * block_w
            pltpu.make_async_copy(
                h_hbm.at[si_v.at[pl.ds(off, block_w)]], hu, sem).wait()
            pltpu.make_async_copy(
                h_hbm.at[di_v.at[pl.ds(off, block_w)]], hv, sem).wait()

        nk = d_feat // _L
        iota16 = lax.iota(jnp.int32, _L)

        def compute(b, hu, hv):
            @pl.loop(0, block_w, step=_L)
            def _(e0):
                edge_sums = []
                for j in range(_L):
                    hu_e = hu.at[e0 + j]
                    hv_e = hv.at[e0 + j]
                    parts = [hu_e[pl.ds(k * _L, _L)] * hv_e[pl.ds(k * _L, _L)]
                             for k in range(nk)]
                    while len(parts) > 1:
                        parts = [parts[i] + parts[i + 1]
                                 for i in range(0, len(parts), 2)]
                    edge_sums.append(parts[0])
                while len(edge_sums) > 1:
                    edge_sums = [edge_sums[i] + edge_sums[i + 1]
                                 for i in range(0, len(edge_sums), 2)]
                sc_v[pl.ds(b * block_w + e0, _L)] = edge_sums[0]

        fire(0, hu0, hv0, sem0)

        @pl.loop(0, nb - 1, step=2)
        def _(b):
            fire(b + 1, hu1, hv1, sem1)
            drain(b, hu0, hv0, sem0)
            compute(b, hu0, hv0)
            fire(b + 2, hu0, hv0, sem0)
            drain(b + 1, hu1, hv1, sem1)
            compute(b + 1, hu1, hv1)

        drain(nb - 1, hu0, hv0, sem0)
        compute(nb - 1, hu0, hv0)
        pltpu.sync_copy(sc_v, out_hbm.at[pl.ds(base, epw)])

    return kern


@jax.jit
def kernel(h, edge_index):
    n_edges = edge_index.shape[1]
    d_feat = h.shape[1]
    ei = edge_index.astype(jnp.int32)
    scores = _edge_dot_kernel(n_edges, d_feat, block_w=80)(h, ei[0], ei[1])
    return scores.reshape(n_edges, 1)


# speedup vs baseline: 1.2918x; 1.2918x over previous
import dataclasses
import functools

import jax
import jax.numpy as jnp
from jax import lax
from jax.experimental import pallas as pl
from jax.experimental.pallas import tpu as pltpu
from jax.experimental.pallas import tpu_sc as plsc

_NC = 2
_NS = 16
_NW = _NC * _NS
_L = 16


def _edge_dot_kernel(n_edges, d_feat, block_w):
    epw = n_edges // _NW
    nb = epw // block_w

    mesh = plsc.VectorSubcoreMesh(core_axis_name="c", subcore_axis_name="s")
    cp = pltpu.CompilerParams()
    if "needs_layout_passes" in pltpu.CompilerParams.__dataclass_fields__:
        cp = dataclasses.replace(cp, needs_layout_passes=False)
    if "use_tc_tiling_on_sc" in pltpu.CompilerParams.__dataclass_fields__:
        cp = dataclasses.replace(cp, use_tc_tiling_on_sc=False)

    @functools.partial(
        pl.kernel,
        mesh=mesh,
        compiler_params=cp,
        out_type=jax.ShapeDtypeStruct((n_edges,), jnp.float32),
        scratch_types=[
            pltpu.VMEM((epw,), jnp.int32),
            pltpu.VMEM((epw,), jnp.int32),
            pltpu.VMEM((epw,), jnp.float32),
            pltpu.VMEM((block_w, d_feat // 2), jnp.float32),
            pltpu.VMEM((block_w, d_feat // 2), jnp.float32),
            pltpu.VMEM((block_w, d_feat // 2), jnp.float32),
            pltpu.VMEM((block_w, d_feat // 2), jnp.float32),
            pltpu.VMEM((_L, _L + 1), jnp.float32),
            pltpu.SemaphoreType.DMA,
            pltpu.SemaphoreType.DMA,
        ],
    )
    def kern(h_hbm, src_hbm, dst_hbm, out_hbm,
             si_v, di_v, sc_v, hu0, hv0, hu1, hv1, tmp_v, sem0, sem1):
        wid = lax.axis_index("s") * _NC + lax.axis_index("c")
        base = wid * epw
        pltpu.sync_copy(src_hbm.at[pl.ds(base, epw)], si_v)
        pltpu.sync_copy(dst_hbm.at[pl.ds(base, epw)], di_v)

        def fire(b, hu, hv, sem):
            off = b * block_w
            pltpu.async_copy(h_hbm.at[si_v.at[pl.ds(off, block_w)]], hu, sem)
            pltpu.async_copy(h_hbm.at[di_v.at[pl.ds(off, block_w)]], hv, sem)

        def drain(b, hu, hv, sem):
            off = b * block_w
            pltpu.make_async_copy(
                h_hbm.at[si_v.at[pl.ds(off, block_w)]], hu, sem).wait()
            pltpu.make_async_copy(
                h_hbm.at[di_v.at[pl.ds(off, block_w)]], hv, sem).wait()

        nk = d_feat // (2 * _L)
        iota16 = lax.iota(jnp.int32, _L)

        def compute(b, hu, hv):
            @pl.loop(0, block_w, step=_L)
            def _(e0):
                for j in range(_L):
                    hu_e = hu.at[e0 + j]
                    hv_e = hv.at[e0 + j]
                    parts = []
                    for k in range(nk):
                        au = plsc.bitcast(hu_e[pl.ds(k * _L, _L)], jnp.bfloat16)
                        av = plsc.bitcast(hv_e[pl.ds(k * _L, _L)], jnp.bfloat16)
                        u0, u1 = plsc.unpack(
                            au, format=plsc.PackFormat.INTERLEAVED,
                            preferred_element_type=jnp.float32)
                        v0, v1 = plsc.unpack(
                            av, format=plsc.PackFormat.INTERLEAVED,
                            preferred_element_type=jnp.float32)
                        parts.append(u0 * v0)
                        parts.append(u1 * v1)
                    while len(parts) > 1:
                        parts = [parts[i] + parts[i + 1]
                                 for i in range(0, len(parts), 2)]
                    plsc.store_scatter(
                        tmp_v, [iota16, jnp.full((_L,), j, jnp.int32)], parts[0])
                rows = [tmp_v[i, pl.ds(0, _L)] for i in range(_L)]
                while len(rows) > 1:
                    rows = [rows[i] + rows[i + 1] for i in range(0, len(rows), 2)]
                sc_v[pl.ds(b * block_w + e0, _L)] = (
                    1.0 / (1.0 + jnp.exp(-rows[0])))

        fire(0, hu0, hv0, sem0)

        @pl.loop(0, nb - 1, step=2)
        def _(b):
            fire(b + 1, hu1, hv1, sem1)
            drain(b, hu0, hv0, sem0)
            compute(b, hu0, hv0)
            fire(b + 2, hu0, hv0, sem0)
            drain(b + 1, hu1, hv1, sem1)
            compute(b + 1, hu1, hv1)

        drain(nb - 1, hu0, hv0, sem0)
        compute(nb - 1, hu0, hv0)
        pltpu.sync_copy(sc_v, out_hbm.at[pl.ds(base, epw)])

    return kern


@jax.jit
def kernel(h, edge_index):
    n_edges = edge_index.shape[1]
    d_feat = h.shape[1]
    ei = edge_index.astype(jnp.int32)
    hb = lax.bitcast_convert_type(
        h.astype(jnp.bfloat16).reshape(h.shape[0], d_feat // 2, 2),
        jnp.float32)
    scores = _edge_dot_kernel(n_edges, d_feat, block_w=80)(hb, ei[0], ei[1])
    return scores.reshape(n_edges, 1)


# speedup vs baseline: 1.3235x; 1.0245x over previous
import dataclasses
import functools

import jax
import jax.numpy as jnp
from jax import lax
from jax.experimental import pallas as pl
from jax.experimental.pallas import tpu as pltpu
from jax.experimental.pallas import tpu_sc as plsc

_NC = 2
_NS = 16
_NW = _NC * _NS
_L = 16


def _edge_dot_kernel(n_edges, d_feat, block_w):
    epw = n_edges // _NW
    nb = epw // block_w

    mesh = plsc.VectorSubcoreMesh(core_axis_name="c", subcore_axis_name="s")
    cp = pltpu.CompilerParams()
    if "needs_layout_passes" in pltpu.CompilerParams.__dataclass_fields__:
        cp = dataclasses.replace(cp, needs_layout_passes=False)
    if "use_tc_tiling_on_sc" in pltpu.CompilerParams.__dataclass_fields__:
        cp = dataclasses.replace(cp, use_tc_tiling_on_sc=False)

    @functools.partial(
        pl.kernel,
        mesh=mesh,
        compiler_params=cp,
        out_type=jax.ShapeDtypeStruct((n_edges,), jnp.float32),
        scratch_types=[
            pltpu.VMEM((epw,), jnp.int32),
            pltpu.VMEM((epw,), jnp.int32),
            pltpu.VMEM((epw,), jnp.float32),
            pltpu.VMEM((block_w, d_feat // 2), jnp.float32),
            pltpu.VMEM((block_w, d_feat // 2), jnp.float32),
            pltpu.VMEM((block_w, d_feat // 2), jnp.float32),
            pltpu.VMEM((block_w, d_feat // 2), jnp.float32),
            pltpu.VMEM((_L, _L + 1), jnp.float32),
            pltpu.SemaphoreType.DMA,
            pltpu.SemaphoreType.DMA,
        ],
    )
    def kern(h_hbm, src_hbm, dst_hbm, out_hbm,
             si_v, di_v, sc_v, hu0, hv0, hu1, hv1, tmp_v, sem0, sem1):
        wid = lax.axis_index("s") * _NC + lax.axis_index("c")
        base = wid * epw
        pltpu.sync_copy(src_hbm.at[pl.ds(base, epw)], si_v)
        pltpu.sync_copy(dst_hbm.at[pl.ds(base, epw)], di_v)

        def fire(b, hu, hv, sem):
            off = b * block_w
            pltpu.async_copy(h_hbm.at[si_v.at[pl.ds(off, block_w)]], hu, sem)
            pltpu.async_copy(h_hbm.at[di_v.at[pl.ds(off, block_w)]], hv, sem)

        def drain(b, hu, hv, sem):
            off = b * block_w
            pltpu.make_async_copy(
                h_hbm.at[si_v.at[pl.ds(off, block_w)]], hu, sem).wait()
            pltpu.make_async_copy(
                h_hbm.at[di_v.at[pl.ds(off, block_w)]], hv, sem).wait()

        nk = d_feat // (2 * _L)
        iota16 = lax.iota(jnp.int32, _L)

        def compute(b, hu, hv):
            @pl.loop(0, block_w, step=_L)
            def _(e0):
                for j in range(_L):
                    hu_e = hu.at[e0 + j]
                    hv_e = hv.at[e0 + j]
                    parts = []
                    for k in range(nk):
                        au = plsc.bitcast(hu_e[pl.ds(k * _L, _L)], jnp.bfloat16)
                        av = plsc.bitcast(hv_e[pl.ds(k * _L, _L)], jnp.bfloat16)
                        p0, p1 = plsc.unpack(
                            au * av, format=plsc.PackFormat.INTERLEAVED,
                            preferred_element_type=jnp.float32)
                        parts.append(p0)
                        parts.append(p1)
                    while len(parts) > 1:
                        parts = [parts[i] + parts[i + 1]
                                 for i in range(0, len(parts), 2)]
                    plsc.store_scatter(
                        tmp_v, [iota16, jnp.full((_L,), j, jnp.int32)], parts[0])
                rows = [tmp_v[i, pl.ds(0, _L)] for i in range(_L)]
                while len(rows) > 1:
                    rows = [rows[i] + rows[i + 1] for i in range(0, len(rows), 2)]
                sc_v[pl.ds(b * block_w + e0, _L)] = (
                    1.0 / (1.0 + jnp.exp(-rows[0])))

        fire(0, hu0, hv0, sem0)

        @pl.loop(0, nb - 1, step=2)
        def _(b):
            fire(b + 1, hu1, hv1, sem1)
            drain(b, hu0, hv0, sem0)
            compute(b, hu0, hv0)
            fire(b + 2, hu0, hv0, sem0)
            drain(b + 1, hu1, hv1, sem1)
            compute(b + 1, hu1, hv1)

        drain(nb - 1, hu0, hv0, sem0)
        compute(nb - 1, hu0, hv0)
        pltpu.sync_copy(sc_v, out_hbm.at[pl.ds(base, epw)])

    return kern


@jax.jit
def kernel(h, edge_index):
    n_edges = edge_index.shape[1]
    d_feat = h.shape[1]
    ei = edge_index.astype(jnp.int32)
    hb = lax.bitcast_convert_type(
        h.astype(jnp.bfloat16).reshape(h.shape[0], d_feat // 2, 2),
        jnp.float32)
    scores = _edge_dot_kernel(n_edges, d_feat, block_w=80)(hb, ei[0], ei[1])
    return scores.reshape(n_edges, 1)
